# baseline (speedup 1.0000x reference)
import functools

import jax
import jax.numpy as jnp
from jax import lax
from jax.experimental import pallas as pl
from jax.experimental.pallas import tpu as pltpu
from jax.experimental.pallas import tpu_sc as plsc

SEQ_LEN = 200
BATCH = 4096
EMBED_DIM = 32
N_CLASSES = 2
VOCAB = 1000000

NC = 2
NS = 16
NW = NC * NS
COLS = BATCH // NW
CHUNK = 100
NCH = SEQ_LEN // CHUNK
LANES = 16
PW = 8
PACK = 128 // PW
BR = 4096
GRID = 16
R = BR * GRID
VP = R * PACK
NBUF = 8


def _compress_body(*refs):
    xrefs = refs[:PACK]
    b2_ref = refs[PACK]
    p_ref = refs[PACK + 1]
    a = jnp.concatenate([x[...] for x in xrefs], axis=0)
    p_ref[...] = jax.lax.dot_general(
        a, b2_ref[...],
        dimension_numbers=(((0,), (0,)), ((), ())),
        preferred_element_type=jnp.float32,
    )


def _compress(embT, B2):
    last_ok = pl.cdiv(VOCAB, BR) - 1
    in_specs = [
        pl.BlockSpec(
            (EMBED_DIM, BR),
            functools.partial(
                lambda q, i: (0, jnp.minimum(q * GRID + i, last_ok)), q))
        for q in range(PACK)
    ]
    in_specs.append(
        pl.BlockSpec((PACK * EMBED_DIM, 128), lambda i: (0, 0)))
    return pl.pallas_call(
        _compress_body,
        grid=(GRID,),
        in_specs=in_specs,
        out_specs=pl.BlockSpec((BR, 128), lambda i: (i, 0)),
        out_shape=jax.ShapeDtypeStruct((R, 128), jnp.float32),
    )(*([embT] * PACK), B2)


def _sc_body(text_hbm, p_hbm, out_hbm, idx_v, *rest):
    bufs = rest[:NBUF]
    acc_v = rest[NBUF]
    sems = rest[NBUF + 1:NBUF + 1 + NBUF]

    wid = lax.axis_index("s") * NC + lax.axis_index("c")
    base = wid * COLS
    pltpu.sync_copy(text_hbm.at[pl.ds(base, COLS)], idx_v)

    def start(c, j, buf, sem):
        pltpu.async_copy(p_hbm.at[idx_v.at[c, j]], buf, sem)

    def wait(buf, sem):
        pltpu.make_async_copy(p_hbm.at[idx_v.at[0, 0]], buf, sem).wait()

    def reduce_into(buf, acc):
        lane = lax.iota(jnp.int32, LANES)
        col = lane % PW
        rpar = lax.shift_right_logical(lane, 3)
        for k in range(CHUNK // 2):
            acc = acc + plsc.load_gather(buf, [2 * k + rpar, col])
        return acc

    NSTEP = COLS * NCH
    for k in range(NBUF - 1):
        start(k >> 1, k & 1, bufs[k], sems[k])

    def group(g, _):
        acc = jnp.zeros((LANES,), jnp.float32)
        for b in range(NBUF):
            s_next = g * NBUF + b + NBUF - 1

            @pl.when(s_next < NSTEP)
            def _(s_next=s_next, b=b):
                start(lax.shift_right_logical(s_next, 1),
                      (b + NBUF - 1) & 1,
                      bufs[(b - 1) % NBUF], sems[(b - 1) % NBUF])

            wait(bufs[b], sems[b])
            if b % 2 == 0:
                acc = reduce_into(bufs[b], jnp.zeros((LANES,), jnp.float32))
            else:
                acc = reduce_into(bufs[b], acc)
                acc_v[g * (NBUF // 2) + b // 2, :] = acc
        return 0

    lax.fori_loop(0, NSTEP // NBUF, group, 0)
    pltpu.sync_copy(acc_v, out_hbm.at[pl.ds(base, COLS)])


def _make_sc_sum():
    mesh = plsc.VectorSubcoreMesh(core_axis_name="c", subcore_axis_name="s")
    return pl.kernel(
        _sc_body,
        mesh=mesh,
        compiler_params=pltpu.CompilerParams(
            use_tc_tiling_on_sc=False, needs_layout_passes=False),
        out_type=jax.ShapeDtypeStruct((BATCH, LANES), jnp.float32),
        scratch_types=(
            [pltpu.VMEM((COLS, NCH, CHUNK), jnp.int32)]
            + [pltpu.VMEM((CHUNK, PW), jnp.float32) for _ in range(NBUF)]
            + [pltpu.VMEM((COLS, LANES), jnp.float32)]
            + [pltpu.SemaphoreType.DMA for _ in range(NBUF)]
        ),
    )


def _head_body(acc_ref, len_ref, b_ref, out_ref):
    a = acc_ref[...]
    s0 = a[:, 0:1] + a[:, 8:9]
    s1 = a[:, 1:2] + a[:, 9:10]
    s = jnp.concatenate([s0, s1], axis=1)
    lenf = len_ref[...].astype(jnp.float32)
    out_ref[...] = s / (lenf + 1.0) + b_ref[...]


def _head(acc, length2d, b2d):
    return pl.pallas_call(
        _head_body,
        out_shape=jax.ShapeDtypeStruct((BATCH, N_CLASSES), jnp.float32),
    )(acc, length2d, b2d)


def kernel(text, length, embeddings, W, b):
    text_packed = (text % R) * PACK + text // R
    textT = jnp.reshape(jnp.transpose(text_packed), (BATCH, NCH, CHUNK))
    embT = jnp.transpose(embeddings)
    Wpad = jnp.concatenate(
        [W, jnp.zeros((EMBED_DIM, PW - N_CLASSES), jnp.float32)], axis=1)
    eye = jnp.eye(PACK, dtype=jnp.float32)
    B2 = jnp.einsum("pq,dc->pdqc", eye, Wpad).reshape(
        PACK * EMBED_DIM, PACK * PW)
    P_pack = _compress(embT, B2)
    P = jnp.reshape(P_pack, (VP, PW))
    acc = _make_sc_sum()(textT, P)
    length2d = jnp.reshape(length, (BATCH, 1))
    b2d = jnp.reshape(b, (1, N_CLASSES))
    return _head(acc, length2d, b2d)


# speedup vs baseline: 1.1020x; 1.1020x over previous
import functools

import jax
import jax.numpy as jnp
from jax import lax
from jax.experimental import pallas as pl
from jax.experimental.pallas import tpu as pltpu
from jax.experimental.pallas import tpu_sc as plsc

SEQ_LEN = 200
BATCH = 4096
EMBED_DIM = 32
N_CLASSES = 2
VOCAB = 1000000

NC = 2
NS = 16
NW = NC * NS
COLS = BATCH // NW
JOFF = (0, 104)
JLEN = (104, 96)
NCH = 2
LANES = 16
PW = 8
PACK = 128 // PW
BR = 4096
GRID = 16
R = BR * GRID
VP = R * PACK
NBUF = 8


def _compress_body(*refs):
    xrefs = refs[:PACK]
    b2_ref = refs[PACK]
    p_ref = refs[PACK + 1]
    a = jnp.concatenate([x[...] for x in xrefs], axis=0)
    p_ref[...] = jax.lax.dot_general(
        a, b2_ref[...],
        dimension_numbers=(((0,), (0,)), ((), ())),
        preferred_element_type=jnp.float32,
    )


def _compress(embT, B2):
    last_ok = pl.cdiv(VOCAB, BR) - 1
    in_specs = [
        pl.BlockSpec(
            (EMBED_DIM, BR),
            functools.partial(
                lambda q, i: (0, jnp.minimum(q * GRID + i, last_ok)), q))
        for q in range(PACK)
    ]
    in_specs.append(
        pl.BlockSpec((PACK * EMBED_DIM, 128), lambda i: (0, 0)))
    return pl.pallas_call(
        _compress_body,
        grid=(GRID,),
        in_specs=in_specs,
        out_specs=pl.BlockSpec((BR, 128), lambda i: (i, 0)),
        out_shape=jax.ShapeDtypeStruct((R, 128), jnp.float32),
    )(*([embT] * PACK), B2)


def _sc_body(text_hbm, p_hbm, out_hbm, idx_v, *rest):
    bufs = rest[:NBUF]
    acc_v = rest[NBUF]
    sems = rest[NBUF + 1:NBUF + 1 + NBUF]

    wid = lax.axis_index("s") * NC + lax.axis_index("c")
    base = wid * COLS
    pltpu.sync_copy(text_hbm.at[pl.ds(base, COLS)], idx_v)

    def start(c, j, buf, sem):
        pltpu.async_copy(
            p_hbm.at[idx_v.at[c, pl.ds(JOFF[j], JLEN[j])]], buf, sem)

    def wait(j, buf, sem):
        pltpu.make_async_copy(
            p_hbm.at[idx_v.at[0, pl.ds(JOFF[j], JLEN[j])]], buf, sem).wait()

    def reduce_into(j, buf, acc):
        lane = lax.iota(jnp.int32, LANES)
        col = lane % PW
        rpar = lax.shift_right_logical(lane, 3)
        for k in range(JLEN[j] // 2):
            acc = acc + plsc.load_gather(buf, [2 * k + rpar, col])
        return acc

    NSTEP = COLS * NCH
    for k in range(NBUF - 1):
        start(k >> 1, k & 1, bufs[k], sems[k])

    def group(g, _):
        acc = jnp.zeros((LANES,), jnp.float32)
        for b in range(NBUF):
            s_next = g * NBUF + b + NBUF - 1

            @pl.when(s_next < NSTEP)
            def _(s_next=s_next, b=b):
                start(lax.shift_right_logical(s_next, 1),
                      (b + NBUF - 1) & 1,
                      bufs[(b - 1) % NBUF], sems[(b - 1) % NBUF])

            wait(b & 1, bufs[b], sems[b])
            if b % 2 == 0:
                acc = reduce_into(0, bufs[b],
                                  jnp.zeros((LANES,), jnp.float32))
            else:
                acc = reduce_into(1, bufs[b], acc)
                acc_v[g * (NBUF // 2) + b // 2, :] = acc
        return 0

    lax.fori_loop(0, NSTEP // NBUF, group, 0)
    pltpu.sync_copy(acc_v, out_hbm.at[pl.ds(base, COLS)])


def _make_sc_sum():
    mesh = plsc.VectorSubcoreMesh(core_axis_name="c", subcore_axis_name="s")
    return pl.kernel(
        _sc_body,
        mesh=mesh,
        compiler_params=pltpu.CompilerParams(
            use_tc_tiling_on_sc=False, needs_layout_passes=False),
        out_type=jax.ShapeDtypeStruct((BATCH, LANES), jnp.float32),
        scratch_types=(
            [pltpu.VMEM((COLS, SEQ_LEN), jnp.int32)]
            + [pltpu.VMEM((JLEN[b & 1], PW), jnp.float32)
               for b in range(NBUF)]
            + [pltpu.VMEM((COLS, LANES), jnp.float32)]
            + [pltpu.SemaphoreType.DMA for _ in range(NBUF)]
        ),
    )


def _head_body(acc_ref, len_ref, b_ref, out_ref):
    a = acc_ref[...]
    s0 = a[:, 0:1] + a[:, 8:9]
    s1 = a[:, 1:2] + a[:, 9:10]
    s = jnp.concatenate([s0, s1], axis=1)
    lenf = len_ref[...].astype(jnp.float32)
    out_ref[...] = s / (lenf + 1.0) + b_ref[...]


def _head(acc, length2d, b2d):
    return pl.pallas_call(
        _head_body,
        out_shape=jax.ShapeDtypeStruct((BATCH, N_CLASSES), jnp.float32),
    )(acc, length2d, b2d)


def kernel(text, length, embeddings, W, b):
    text_packed = (text % R) * PACK + text // R
    textT = jnp.transpose(text_packed)
    embT = jnp.transpose(embeddings)
    Wpad = jnp.concatenate(
        [W, jnp.zeros((EMBED_DIM, PW - N_CLASSES), jnp.float32)], axis=1)
    eye = jnp.eye(PACK, dtype=jnp.float32)
    B2 = jnp.einsum("pq,dc->pdqc", eye, Wpad).reshape(
        PACK * EMBED_DIM, PACK * PW)
    P_pack = _compress(embT, B2)
    P = jnp.reshape(P_pack, (VP, PW))
    acc = _make_sc_sum()(textT, P)
    length2d = jnp.reshape(length, (BATCH, 1))
    b2d = jnp.reshape(b, (1, N_CLASSES))
    return _head(acc, length2d, b2d)
